# baseline (speedup 1.0000x reference)
import functools
import jax
import jax.numpy as jnp
from jax import lax
from jax.experimental import pallas as pl
from jax.experimental.pallas import tpu as pltpu
from jax.experimental.pallas import tpu_sc as plsc

N = 10000
D = 128
G = 16
NC = 2
NS = 16
NW = NC * NS
NP = 10240
E0 = 320000
EL = E0 + N
ET = 10320
EP = ET * NW
CC = 80
NCH = ET // CC
NSL = NP // NS
RPT = NP // NS
ZR = 128
BLK = 1000


def _rsqrt16(d):
    i = plsc.bitcast(d, jnp.int32)
    i = jnp.int32(0x5F3759DF) - lax.shift_right_logical(i, jnp.int32(1))
    y = plsc.bitcast(i, jnp.float32)
    for _ in range(3):
        y = y * (1.5 - (0.5 * d) * y * y)
    return y


def _sc_mesh():
    return plsc.VectorSubcoreMesh(core_axis_name="c", subcore_axis_name="s",
                                  num_cores=NC, num_subcores=NS)


@functools.cache
def _build_sc_prep():
    @functools.partial(
        pl.kernel,
        out_type=jax.ShapeDtypeStruct((EP,), jnp.float32),
        mesh=_sc_mesh(),
        compiler_params=pltpu.CompilerParams(needs_layout_passes=False),
        scratch_types=[
            pltpu.VMEM((ET,), jnp.int32),
            pltpu.VMEM((ET,), jnp.float32),
            pltpu.VMEM((ET,), jnp.int32),
            pltpu.VMEM((ET,), jnp.float32),
            pltpu.VMEM((NP,), jnp.float32),
            pltpu.VMEM((NP,), jnp.float32),
            pltpu.VMEM_SHARED((NS, NP), jnp.float32),
            pltpu.VMEM_SHARED((NP,), jnp.float32),
        ],
    )
    def sc_prep(row_hbm, col_hbm, ew_hbm, norm_hbm,
                idxb, ewb, idx2b, normb, dega, dinvb, deg_sh, dinv_sh):
        c = lax.axis_index("c")
        s = lax.axis_index("s")
        wid = s * NC + c

        def zbody(i, _):
            dega[pl.ds(i * 16, 16)] = jnp.zeros((16,), jnp.float32)
            return 0
        lax.fori_loop(0, NP // 16, zbody, 0)

        for slab in (s, s + NS):
            pltpu.sync_copy(col_hbm.at[pl.ds(slab * ET, ET)], idxb)
            pltpu.sync_copy(ew_hbm.at[pl.ds(slab * ET, ET)], ewb)

            def dbody(j, _):
                sl = pl.ds(j * 16, 16)
                plsc.addupdate_scatter(dega, [idxb[sl]], ewb[sl])
                return 0
            lax.fori_loop(0, ET // 16, dbody, 0)

        pltpu.sync_copy(dega, deg_sh.at[s])
        plsc.subcore_barrier()

        base = s * NSL
        pltpu.sync_copy(deg_sh.at[0, pl.ds(base, NSL)], dega.at[pl.ds(0, NSL)])

        def rbody(p, _):
            pltpu.sync_copy(deg_sh.at[p, pl.ds(base, NSL)],
                            ewb.at[pl.ds(0, NSL)])

            def abody(j, _):
                sl = pl.ds(j * 16, 16)
                dega[sl] = dega[sl] + ewb[sl]
                return 0
            lax.fori_loop(0, NSL // 16, abody, 0)
            return 0
        lax.fori_loop(1, NS, rbody, 0)

        def dibody(j, _):
            sl = pl.ds(j * 16, 16)
            dinvb[sl] = _rsqrt16(dega[sl])
            return 0
        lax.fori_loop(0, NSL // 16, dibody, 0)

        pltpu.sync_copy(dinvb.at[pl.ds(0, NSL)], dinv_sh.at[pl.ds(base, NSL)])
        plsc.subcore_barrier()
        pltpu.sync_copy(dinv_sh, dinvb)

        ebase = wid * ET
        pltpu.sync_copy(row_hbm.at[pl.ds(ebase, ET)], idx2b)
        pltpu.sync_copy(col_hbm.at[pl.ds(ebase, ET)], idxb)
        pltpu.sync_copy(ew_hbm.at[pl.ds(ebase, ET)], ewb)

        def nbody(j, _):
            sl = pl.ds(j * 16, 16)
            dr = plsc.load_gather(dinvb, [idx2b[sl]])
            dc = plsc.load_gather(dinvb, [idxb[sl]])
            normb[sl] = dr * ewb[sl] * dc
            return 0
        lax.fori_loop(0, ET // 16, nbody, 0)
        pltpu.sync_copy(normb, norm_hbm.at[pl.ds(ebase, ET)])

    return sc_prep


@functools.cache
def _build_sc_edge():
    @functools.partial(
        pl.kernel,
        out_type=jax.ShapeDtypeStruct((NC, NP, D), jnp.float32),
        mesh=_sc_mesh(),
        compiler_params=pltpu.CompilerParams(needs_layout_passes=False),
        scratch_types=[
            pltpu.VMEM((ET,), jnp.int32),
            pltpu.VMEM((2, CC), jnp.int32),
            pltpu.VMEM((2, CC), jnp.float32),
            pltpu.VMEM((CC, D), jnp.float32),
            pltpu.VMEM((CC, D), jnp.float32),
            pltpu.VMEM_SHARED((NP, D), jnp.float32),
            pltpu.SemaphoreType.DMA,
            pltpu.SemaphoreType.DMA,
            pltpu.SemaphoreType.DMA,
            pltpu.SemaphoreType.DMA,
        ],
    )
    def sc_edge(xw_hbm, row_hbm, col_hbm, norm_hbm, out_hbm,
                rowb, colcb, normcb, buf0, buf1, acc_sh,
                sem0, sem1, ssem0, ssem1):
        c = lax.axis_index("c")
        s = lax.axis_index("s")
        wid = s * NC + c

        def zb(i, _):
            def zb2(j, _):
                buf0[i, pl.ds(j * 16, 16)] = jnp.zeros((16,), jnp.float32)
                return 0
            lax.fori_loop(0, D // 16, zb2, 0)
            return 0
        lax.fori_loop(0, CC, zb, 0)
        rbase = s * RPT
        for q in range(RPT // CC):
            pltpu.sync_copy(buf0, acc_sh.at[pl.ds(rbase + q * CC, CC)])
        zrem = RPT - (RPT // CC) * CC
        if zrem:
            pltpu.sync_copy(buf0.at[pl.ds(0, zrem)],
                            acc_sh.at[pl.ds(rbase + RPT - zrem, zrem)])

        ebase = wid * ET
        pltpu.sync_copy(row_hbm.at[pl.ds(ebase, ET)], rowb)
        plsc.subcore_barrier()

        SS = CC // 16
        SREM = CC - SS * 16

        def gstart(k, b, bufr, sem):
            pltpu.async_copy(xw_hbm.at[rowb.at[pl.ds(k * CC, CC)]], bufr, sem)
            pltpu.async_copy(col_hbm.at[pl.ds(ebase + k * CC, CC)],
                             colcb.at[b], sem)
            pltpu.async_copy(norm_hbm.at[pl.ds(ebase + k * CC, CC)],
                             normcb.at[b], sem)

        def gwait(k, b, bufr, sem):
            pltpu.make_async_copy(xw_hbm.at[rowb.at[pl.ds(k * CC, CC)]],
                                  bufr, sem).wait()
            pltpu.make_async_copy(col_hbm.at[pl.ds(ebase + k * CC, CC)],
                                  colcb.at[b], sem).wait()
            pltpu.make_async_copy(norm_hbm.at[pl.ds(ebase + k * CC, CC)],
                                  normcb.at[b], sem).wait()

        def scale(b, bufr):
            def mulrow(i, sc):
                for j in range(D // 16):
                    sl = pl.ds(j * 16, 16)
                    bufr[i, sl] = bufr[i, sl] * sc

            def edge16(g, _):
                nv = normcb[b, pl.ds(g * 16, 16)]
                for e in range(16):
                    mulrow(g * 16 + e, nv[e])
                return 0
            lax.fori_loop(0, SS, edge16, 0)
            if SREM:
                nv = normcb[b, pl.ds(CC - 16, 16)]
                for e in range(16 - SREM, 16):
                    mulrow(CC - 16 + e, nv[e])

        def sstart(b, bufr, ssem):
            pltpu.async_copy(bufr, acc_sh.at[colcb.at[b]], ssem, add=True)

        def swait(b, bufr, ssem):
            pltpu.make_async_copy(bufr, acc_sh.at[colcb.at[b]], ssem).wait()

        gstart(0, 0, buf0, sem0)

        def pairs(t, _):
            k0 = 2 * t

            @pl.when(t > 0)
            def _():
                swait(1, buf1, ssem1)
            gstart(k0 + 1, 1, buf1, sem1)
            gwait(k0, 0, buf0, sem0)
            scale(0, buf0)
            sstart(0, buf0, ssem0)
            gwait(k0 + 1, 1, buf1, sem1)
            scale(1, buf1)
            sstart(1, buf1, ssem1)
            swait(0, buf0, ssem0)

            @pl.when(k0 + 2 < NCH)
            def _():
                gstart(k0 + 2, 0, buf0, sem0)
            return 0
        lax.fori_loop(0, NCH // 2, pairs, 0)
        if NCH % 2:
            gwait(NCH - 1, 0, buf0, sem0)
            scale(0, buf0)
            sstart(0, buf0, ssem0)
            swait(0, buf0, ssem0)
        swait(1, buf1, ssem1)

        plsc.subcore_barrier()
        pltpu.sync_copy(acc_sh.at[pl.ds(rbase, RPT)],
                        out_hbm.at[c, pl.ds(rbase, RPT)])

    return sc_edge


def _mm_body(x_ref, w_ref, o_ref):
    o_ref[...] = jnp.dot(x_ref[...], w_ref[...],
                         preferred_element_type=jnp.float32)


def _tc_matmul(xx, ww):
    n, d = xx.shape
    h = ww.shape[1]
    return pl.pallas_call(
        _mm_body,
        grid=(n // BLK,),
        in_specs=[pl.BlockSpec((BLK, d), lambda i: (i, 0)),
                  pl.BlockSpec((d, h), lambda i: (0, 0))],
        out_specs=pl.BlockSpec((BLK, h), lambda i: (i, 0)),
        out_shape=jax.ShapeDtypeStruct((n, h), jnp.float32),
    )(xx, ww)


def _tc2_body(acc_ref, b_ref, w_ref, o_ref):
    t = acc_ref[0] + acc_ref[1] + b_ref[...]
    h = jnp.where(t > 0, t, jnp.exp(jnp.minimum(t, 0.0)) - 1.0)
    o_ref[...] = jnp.dot(h, w_ref[...], preferred_element_type=jnp.float32)


def _tc2(acc, b, ww):
    return pl.pallas_call(
        _tc2_body,
        grid=(N // BLK,),
        in_specs=[pl.BlockSpec((NC, BLK, D), lambda i: (0, i, 0)),
                  pl.BlockSpec((1, D), lambda i: (0, 0)),
                  pl.BlockSpec((D, D), lambda i: (0, 0))],
        out_specs=pl.BlockSpec((BLK, D), lambda i: (i, 0)),
        out_shape=jax.ShapeDtypeStruct((N, D), jnp.float32),
    )(acc, b, ww)


def _tc3_body(acc_ref, b_ref, batch_ref, fcw_ref, fcb_ref, o_ref, psum, pcnt):
    i = pl.program_id(0)

    @pl.when(i == 0)
    def _():
        psum[...] = jnp.zeros_like(psum)
        pcnt[...] = jnp.zeros_like(pcnt)

    t = acc_ref[0] + acc_ref[1] + b_ref[...]
    gids = lax.broadcasted_iota(jnp.int32, (G, BLK), 0)
    onehot = (gids == batch_ref[0]).astype(jnp.float32)
    psum[...] += jnp.dot(onehot, t, preferred_element_type=jnp.float32)
    pcnt[...] += jnp.sum(onehot, axis=1, keepdims=True)

    @pl.when(i == pl.num_programs(0) - 1)
    def _():
        pooled = psum[...] / jnp.maximum(pcnt[...], 1.0)
        o_ref[...] = (jnp.dot(pooled, fcw_ref[...],
                              preferred_element_type=jnp.float32)
                      + fcb_ref[...])


def _tc3(acc, b, batch2, fcwp, fcbp):
    return pl.pallas_call(
        _tc3_body,
        grid=(N // BLK,),
        in_specs=[pl.BlockSpec((NC, BLK, D), lambda i: (0, i, 0)),
                  pl.BlockSpec((1, D), lambda i: (0, 0)),
                  pl.BlockSpec((1, 1, BLK), lambda i: (i, 0, 0)),
                  pl.BlockSpec((D, D), lambda i: (0, 0)),
                  pl.BlockSpec((1, D), lambda i: (0, 0))],
        out_specs=pl.BlockSpec((G, D), lambda i: (0, 0)),
        out_shape=jax.ShapeDtypeStruct((G, D), jnp.float32),
        scratch_shapes=[pltpu.VMEM((G, D), jnp.float32),
                        pltpu.VMEM((G, 1), jnp.float32)],
    )(acc, b, batch2, fcwp, fcbp)


def kernel(x, edge_index, edge_weight, batch, W1, b1, W2, b2, fc_W, fc_b):
    row = edge_index[0].astype(jnp.int32)
    col = edge_index[1].astype(jnp.int32)
    loop = jnp.arange(N, dtype=jnp.int32)
    padi = jnp.zeros((EP - EL,), jnp.int32)
    row_f = jnp.concatenate([row, loop, padi])
    col_f = jnp.concatenate([col, loop, padi])
    ew_f = jnp.concatenate([edge_weight.astype(jnp.float32),
                            jnp.ones((N,), jnp.float32),
                            jnp.zeros((EP - EL,), jnp.float32)])
    sc_prep = _build_sc_prep()
    sc_edge = _build_sc_edge()

    norm = sc_prep(row_f, col_f, ew_f)
    xw1 = _tc_matmul(x, W1)
    acc1 = sc_edge(xw1, row_f, col_f, norm)
    xw2 = _tc2(acc1, b1.reshape(1, D), W2)
    acc2 = sc_edge(xw2, row_f, col_f, norm)

    fcwp = jnp.pad(fc_W, ((0, 0), (0, D - 1)))
    fcbp = jnp.pad(fc_b.reshape(1, 1), ((0, 0), (0, D - 1)))
    out = _tc3(acc2, b2.reshape(1, D),
               batch.astype(jnp.int32).reshape(N // BLK, 1, BLK), fcwp, fcbp)
    return out[:, :1]


# speedup vs baseline: 1.1313x; 1.1313x over previous
import functools
import jax
import jax.numpy as jnp
from jax import lax
from jax.experimental import pallas as pl
from jax.experimental.pallas import tpu as pltpu
from jax.experimental.pallas import tpu_sc as plsc

N = 10000
D = 128
G = 16
NC = 2
NS = 16
NW = NC * NS
NP = 10240
E0 = 320000
EL = E0 + N
ET = 10320
EP = ET * NW
CC = 120
NCH = ET // CC
NSL = NP // NS
RPT = NP // NS
ZR = 128
BLK = 1000


def _rsqrt16(d):
    i = plsc.bitcast(d, jnp.int32)
    i = jnp.int32(0x5F3759DF) - lax.shift_right_logical(i, jnp.int32(1))
    y = plsc.bitcast(i, jnp.float32)
    for _ in range(3):
        y = y * (1.5 - (0.5 * d) * y * y)
    return y


def _sc_mesh():
    return plsc.VectorSubcoreMesh(core_axis_name="c", subcore_axis_name="s",
                                  num_cores=NC, num_subcores=NS)


@functools.cache
def _build_sc_prep():
    @functools.partial(
        pl.kernel,
        out_type=jax.ShapeDtypeStruct((EP,), jnp.float32),
        mesh=_sc_mesh(),
        compiler_params=pltpu.CompilerParams(needs_layout_passes=False),
        scratch_types=[
            pltpu.VMEM((ET,), jnp.int32),
            pltpu.VMEM((ET,), jnp.float32),
            pltpu.VMEM((ET,), jnp.int32),
            pltpu.VMEM((ET,), jnp.float32),
            pltpu.VMEM((NP,), jnp.float32),
            pltpu.VMEM((NP,), jnp.float32),
            pltpu.VMEM_SHARED((NS, NP), jnp.float32),
            pltpu.VMEM_SHARED((NP,), jnp.float32),
        ],
    )
    def sc_prep(row_hbm, col_hbm, ew_hbm, norm_hbm,
                idxb, ewb, idx2b, normb, dega, dinvb, deg_sh, dinv_sh):
        c = lax.axis_index("c")
        s = lax.axis_index("s")
        wid = s * NC + c

        def zbody(i, _):
            dega[pl.ds(i * 16, 16)] = jnp.zeros((16,), jnp.float32)
            return 0
        lax.fori_loop(0, NP // 16, zbody, 0)

        for slab in (s, s + NS):
            pltpu.sync_copy(col_hbm.at[pl.ds(slab * ET, ET)], idxb)
            pltpu.sync_copy(ew_hbm.at[pl.ds(slab * ET, ET)], ewb)

            def dbody(j, _):
                sl = pl.ds(j * 16, 16)
                plsc.addupdate_scatter(dega, [idxb[sl]], ewb[sl])
                return 0
            lax.fori_loop(0, ET // 16, dbody, 0)

        pltpu.sync_copy(dega, deg_sh.at[s])
        plsc.subcore_barrier()

        base = s * NSL
        pltpu.sync_copy(deg_sh.at[0, pl.ds(base, NSL)], dega.at[pl.ds(0, NSL)])

        def rbody(p, _):
            pltpu.sync_copy(deg_sh.at[p, pl.ds(base, NSL)],
                            ewb.at[pl.ds(0, NSL)])

            def abody(j, _):
                sl = pl.ds(j * 16, 16)
                dega[sl] = dega[sl] + ewb[sl]
                return 0
            lax.fori_loop(0, NSL // 16, abody, 0)
            return 0
        lax.fori_loop(1, NS, rbody, 0)

        def dibody(j, _):
            sl = pl.ds(j * 16, 16)
            dinvb[sl] = _rsqrt16(dega[sl])
            return 0
        lax.fori_loop(0, NSL // 16, dibody, 0)

        pltpu.sync_copy(dinvb.at[pl.ds(0, NSL)], dinv_sh.at[pl.ds(base, NSL)])
        plsc.subcore_barrier()
        pltpu.sync_copy(dinv_sh, dinvb)

        ebase = wid * ET
        pltpu.sync_copy(row_hbm.at[pl.ds(ebase, ET)], idx2b)
        pltpu.sync_copy(col_hbm.at[pl.ds(ebase, ET)], idxb)
        pltpu.sync_copy(ew_hbm.at[pl.ds(ebase, ET)], ewb)

        def nbody(j, _):
            sl = pl.ds(j * 16, 16)
            dr = plsc.load_gather(dinvb, [idx2b[sl]])
            dc = plsc.load_gather(dinvb, [idxb[sl]])
            normb[sl] = dr * ewb[sl] * dc
            return 0
        lax.fori_loop(0, ET // 16, nbody, 0)
        pltpu.sync_copy(normb, norm_hbm.at[pl.ds(ebase, ET)])

    return sc_prep


@functools.cache
def _build_sc_edge():
    @functools.partial(
        pl.kernel,
        out_type=jax.ShapeDtypeStruct((NC, NP, D), jnp.float32),
        mesh=_sc_mesh(),
        compiler_params=pltpu.CompilerParams(needs_layout_passes=False),
        scratch_types=[
            pltpu.VMEM((ET,), jnp.int32),
            pltpu.VMEM((2, CC), jnp.int32),
            pltpu.VMEM((2, CC), jnp.float32),
            pltpu.VMEM((CC, D), jnp.float32),
            pltpu.VMEM((CC, D), jnp.float32),
            pltpu.VMEM_SHARED((NP, D), jnp.float32),
            pltpu.SemaphoreType.DMA,
            pltpu.SemaphoreType.DMA,
            pltpu.SemaphoreType.DMA,
            pltpu.SemaphoreType.DMA,
        ],
    )
    def sc_edge(xw_hbm, row_hbm, col_hbm, norm_hbm, out_hbm,
                rowb, colcb, normcb, buf0, buf1, acc_sh,
                sem0, sem1, ssem0, ssem1):
        c = lax.axis_index("c")
        s = lax.axis_index("s")
        wid = s * NC + c

        def zb(i, _):
            def zb2(j, _):
                buf0[i, pl.ds(j * 16, 16)] = jnp.zeros((16,), jnp.float32)
                return 0
            lax.fori_loop(0, D // 16, zb2, 0)
            return 0
        lax.fori_loop(0, CC, zb, 0)
        rbase = s * RPT
        for q in range(RPT // CC):
            pltpu.sync_copy(buf0, acc_sh.at[pl.ds(rbase + q * CC, CC)])
        zrem = RPT - (RPT // CC) * CC
        if zrem:
            pltpu.sync_copy(buf0.at[pl.ds(0, zrem)],
                            acc_sh.at[pl.ds(rbase + RPT - zrem, zrem)])

        ebase = wid * ET
        pltpu.sync_copy(row_hbm.at[pl.ds(ebase, ET)], rowb)
        plsc.subcore_barrier()

        SS = CC // 16
        SREM = CC - SS * 16

        def gstart(k, b, bufr, sem):
            pltpu.async_copy(xw_hbm.at[rowb.at[pl.ds(k * CC, CC)]], bufr, sem)
            pltpu.async_copy(col_hbm.at[pl.ds(ebase + k * CC, CC)],
                             colcb.at[b], sem)
            pltpu.async_copy(norm_hbm.at[pl.ds(ebase + k * CC, CC)],
                             normcb.at[b], sem)

        def gwait(k, b, bufr, sem):
            pltpu.make_async_copy(xw_hbm.at[rowb.at[pl.ds(k * CC, CC)]],
                                  bufr, sem).wait()
            pltpu.make_async_copy(col_hbm.at[pl.ds(ebase + k * CC, CC)],
                                  colcb.at[b], sem).wait()
            pltpu.make_async_copy(norm_hbm.at[pl.ds(ebase + k * CC, CC)],
                                  normcb.at[b], sem).wait()

        def scale(b, bufr):
            def mulrow(i, sc):
                for j in range(D // 16):
                    sl = pl.ds(j * 16, 16)
                    bufr[i, sl] = bufr[i, sl] * sc

            def edge16(g, _):
                nv = normcb[b, pl.ds(g * 16, 16)]
                for e in range(16):
                    mulrow(g * 16 + e, nv[e])
                return 0
            lax.fori_loop(0, SS, edge16, 0)
            if SREM:
                nv = normcb[b, pl.ds(CC - 16, 16)]
                for e in range(16 - SREM, 16):
                    mulrow(CC - 16 + e, nv[e])

        def scatter(b, bufr):
            pltpu.sync_copy(bufr, acc_sh.at[colcb.at[b]], add=True)

        gstart(0, 0, buf0, sem0)

        def pairs(t, _):
            k0 = 2 * t
            gstart(k0 + 1, 1, buf1, sem1)
            gwait(k0, 0, buf0, sem0)
            scale(0, buf0)
            scatter(0, buf0)

            @pl.when(k0 + 2 < NCH)
            def _():
                gstart(k0 + 2, 0, buf0, sem0)
            gwait(k0 + 1, 1, buf1, sem1)
            scale(1, buf1)
            scatter(1, buf1)
            return 0
        lax.fori_loop(0, NCH // 2, pairs, 0)
        if NCH % 2:
            gwait(NCH - 1, 0, buf0, sem0)
            scale(0, buf0)
            scatter(0, buf0)

        plsc.subcore_barrier()
        pltpu.sync_copy(acc_sh.at[pl.ds(rbase, RPT)],
                        out_hbm.at[c, pl.ds(rbase, RPT)])

    return sc_edge


def _mm_body(x_ref, w_ref, o_ref):
    o_ref[...] = jnp.dot(x_ref[...], w_ref[...],
                         preferred_element_type=jnp.float32)


def _tc_matmul(xx, ww):
    n, d = xx.shape
    h = ww.shape[1]
    return pl.pallas_call(
        _mm_body,
        grid=(n // BLK,),
        in_specs=[pl.BlockSpec((BLK, d), lambda i: (i, 0)),
                  pl.BlockSpec((d, h), lambda i: (0, 0))],
        out_specs=pl.BlockSpec((BLK, h), lambda i: (i, 0)),
        out_shape=jax.ShapeDtypeStruct((n, h), jnp.float32),
    )(xx, ww)


def _tc2_body(acc_ref, b_ref, w_ref, o_ref):
    t = acc_ref[0] + acc_ref[1] + b_ref[...]
    h = jnp.where(t > 0, t, jnp.exp(jnp.minimum(t, 0.0)) - 1.0)
    o_ref[...] = jnp.dot(h, w_ref[...], preferred_element_type=jnp.float32)


def _tc2(acc, b, ww):
    return pl.pallas_call(
        _tc2_body,
        grid=(N // BLK,),
        in_specs=[pl.BlockSpec((NC, BLK, D), lambda i: (0, i, 0)),
                  pl.BlockSpec((1, D), lambda i: (0, 0)),
                  pl.BlockSpec((D, D), lambda i: (0, 0))],
        out_specs=pl.BlockSpec((BLK, D), lambda i: (i, 0)),
        out_shape=jax.ShapeDtypeStruct((N, D), jnp.float32),
    )(acc, b, ww)


def _tc3_body(acc_ref, b_ref, batch_ref, fcw_ref, fcb_ref, o_ref, psum, pcnt):
    i = pl.program_id(0)

    @pl.when(i == 0)
    def _():
        psum[...] = jnp.zeros_like(psum)
        pcnt[...] = jnp.zeros_like(pcnt)

    t = acc_ref[0] + acc_ref[1] + b_ref[...]
    gids = lax.broadcasted_iota(jnp.int32, (G, BLK), 0)
    onehot = (gids == batch_ref[0]).astype(jnp.float32)
    psum[...] += jnp.dot(onehot, t, preferred_element_type=jnp.float32)
    pcnt[...] += jnp.sum(onehot, axis=1, keepdims=True)

    @pl.when(i == pl.num_programs(0) - 1)
    def _():
        pooled = psum[...] / jnp.maximum(pcnt[...], 1.0)
        o_ref[...] = (jnp.dot(pooled, fcw_ref[...],
                              preferred_element_type=jnp.float32)
                      + fcb_ref[...])


def _tc3(acc, b, batch2, fcwp, fcbp):
    return pl.pallas_call(
        _tc3_body,
        grid=(N // BLK,),
        in_specs=[pl.BlockSpec((NC, BLK, D), lambda i: (0, i, 0)),
                  pl.BlockSpec((1, D), lambda i: (0, 0)),
                  pl.BlockSpec((1, 1, BLK), lambda i: (i, 0, 0)),
                  pl.BlockSpec((D, D), lambda i: (0, 0)),
                  pl.BlockSpec((1, D), lambda i: (0, 0))],
        out_specs=pl.BlockSpec((G, D), lambda i: (0, 0)),
        out_shape=jax.ShapeDtypeStruct((G, D), jnp.float32),
        scratch_shapes=[pltpu.VMEM((G, D), jnp.float32),
                        pltpu.VMEM((G, 1), jnp.float32)],
    )(acc, b, batch2, fcwp, fcbp)


def kernel(x, edge_index, edge_weight, batch, W1, b1, W2, b2, fc_W, fc_b):
    row = edge_index[0].astype(jnp.int32)
    col = edge_index[1].astype(jnp.int32)
    loop = jnp.arange(N, dtype=jnp.int32)
    padi = jnp.zeros((EP - EL,), jnp.int32)
    row_f = jnp.concatenate([row, loop, padi])
    col_f = jnp.concatenate([col, loop, padi])
    ew_f = jnp.concatenate([edge_weight.astype(jnp.float32),
                            jnp.ones((N,), jnp.float32),
                            jnp.zeros((EP - EL,), jnp.float32)])
    sc_prep = _build_sc_prep()
    sc_edge = _build_sc_edge()

    norm = sc_prep(row_f, col_f, ew_f)
    xw1 = _tc_matmul(x, W1)
    acc1 = sc_edge(xw1, row_f, col_f, norm)
    xw2 = _tc2(acc1, b1.reshape(1, D), W2)
    acc2 = sc_edge(xw2, row_f, col_f, norm)

    fcwp = jnp.pad(fc_W, ((0, 0), (0, D - 1)))
    fcbp = jnp.pad(fc_b.reshape(1, 1), ((0, 0), (0, D - 1)))
    out = _tc3(acc2, b2.reshape(1, D),
               batch.astype(jnp.int32).reshape(N // BLK, 1, BLK), fcwp, fcbp)
    return out[:, :1]


# speedup vs baseline: 1.1558x; 1.0217x over previous
import functools
import jax
import jax.numpy as jnp
from jax import lax
from jax.experimental import pallas as pl
from jax.experimental.pallas import tpu as pltpu
from jax.experimental.pallas import tpu_sc as plsc

N = 10000
D = 128
G = 16
NC = 2
NS = 16
NW = NC * NS
NP = 10240
E0 = 320000
EL = E0 + N
ET = 10320
EP = ET * NW
CC = 80
NCH = ET // CC
NSL = NP // NS
RPT = NP // NS
ZR = 128
BLK = 1000


def _rsqrt16(d):
    i = plsc.bitcast(d, jnp.int32)
    i = jnp.int32(0x5F3759DF) - lax.shift_right_logical(i, jnp.int32(1))
    y = plsc.bitcast(i, jnp.float32)
    for _ in range(3):
        y = y * (1.5 - (0.5 * d) * y * y)
    return y


def _sc_mesh():
    return plsc.VectorSubcoreMesh(core_axis_name="c", subcore_axis_name="s",
                                  num_cores=NC, num_subcores=NS)


@functools.cache
def _build_sc_prep():
    @functools.partial(
        pl.kernel,
        out_type=jax.ShapeDtypeStruct((EP,), jnp.float32),
        mesh=_sc_mesh(),
        compiler_params=pltpu.CompilerParams(needs_layout_passes=False),
        scratch_types=[
            pltpu.VMEM((ET,), jnp.int32),
            pltpu.VMEM((ET,), jnp.float32),
            pltpu.VMEM((ET,), jnp.int32),
            pltpu.VMEM((ET,), jnp.float32),
            pltpu.VMEM((NP,), jnp.float32),
            pltpu.VMEM((NP,), jnp.float32),
            pltpu.VMEM_SHARED((NS, NP), jnp.float32),
            pltpu.VMEM_SHARED((NP,), jnp.float32),
        ],
    )
    def sc_prep(row_hbm, col_hbm, ew_hbm, norm_hbm,
                idxb, ewb, idx2b, normb, dega, dinvb, deg_sh, dinv_sh):
        c = lax.axis_index("c")
        s = lax.axis_index("s")
        wid = s * NC + c

        def zbody(i, _):
            dega[pl.ds(i * 16, 16)] = jnp.zeros((16,), jnp.float32)
            return 0
        lax.fori_loop(0, NP // 16, zbody, 0)

        for slab in (s, s + NS):
            pltpu.sync_copy(col_hbm.at[pl.ds(slab * ET, ET)], idxb)
            pltpu.sync_copy(ew_hbm.at[pl.ds(slab * ET, ET)], ewb)

            def dbody(j, _):
                sl = pl.ds(j * 16, 16)
                plsc.addupdate_scatter(dega, [idxb[sl]], ewb[sl])
                return 0
            lax.fori_loop(0, ET // 16, dbody, 0)

        pltpu.sync_copy(dega, deg_sh.at[s])
        plsc.subcore_barrier()

        base = s * NSL
        pltpu.sync_copy(deg_sh.at[0, pl.ds(base, NSL)], dega.at[pl.ds(0, NSL)])

        def rbody(p, _):
            pltpu.sync_copy(deg_sh.at[p, pl.ds(base, NSL)],
                            ewb.at[pl.ds(0, NSL)])

            def abody(j, _):
                sl = pl.ds(j * 16, 16)
                dega[sl] = dega[sl] + ewb[sl]
                return 0
            lax.fori_loop(0, NSL // 16, abody, 0)
            return 0
        lax.fori_loop(1, NS, rbody, 0)

        def dibody(j, _):
            sl = pl.ds(j * 16, 16)
            dinvb[sl] = _rsqrt16(dega[sl])
            return 0
        lax.fori_loop(0, NSL // 16, dibody, 0)

        pltpu.sync_copy(dinvb.at[pl.ds(0, NSL)], dinv_sh.at[pl.ds(base, NSL)])
        plsc.subcore_barrier()
        pltpu.sync_copy(dinv_sh, dinvb)

        ebase = wid * ET
        pltpu.sync_copy(row_hbm.at[pl.ds(ebase, ET)], idx2b)
        pltpu.sync_copy(col_hbm.at[pl.ds(ebase, ET)], idxb)
        pltpu.sync_copy(ew_hbm.at[pl.ds(ebase, ET)], ewb)

        def nbody(j, _):
            sl = pl.ds(j * 16, 16)
            dr = plsc.load_gather(dinvb, [idx2b[sl]])
            dc = plsc.load_gather(dinvb, [idxb[sl]])
            normb[sl] = dr * ewb[sl] * dc
            return 0
        lax.fori_loop(0, ET // 16, nbody, 0)
        pltpu.sync_copy(normb, norm_hbm.at[pl.ds(ebase, ET)])

    return sc_prep


@functools.cache
def _build_sc_edge():
    @functools.partial(
        pl.kernel,
        out_type=jax.ShapeDtypeStruct((NC, NP, D), jnp.float32),
        mesh=_sc_mesh(),
        compiler_params=pltpu.CompilerParams(needs_layout_passes=False),
        scratch_types=[
            pltpu.VMEM((ET,), jnp.int32),
            pltpu.VMEM((3, CC), jnp.int32),
            pltpu.VMEM((3, CC), jnp.float32),
            pltpu.VMEM((CC, D), jnp.float32),
            pltpu.VMEM((CC, D), jnp.float32),
            pltpu.VMEM((CC, D), jnp.float32),
            pltpu.VMEM_SHARED((NP, D), jnp.float32),
            pltpu.SemaphoreType.DMA,
            pltpu.SemaphoreType.DMA,
            pltpu.SemaphoreType.DMA,
        ],
    )
    def sc_edge(xw_hbm, row_hbm, col_hbm, norm_hbm, out_hbm,
                rowb, colcb, normcb, buf0, buf1, buf2, acc_sh,
                sem0, sem1, sem2):
        c = lax.axis_index("c")
        s = lax.axis_index("s")
        wid = s * NC + c

        def zb(i, _):
            def zb2(j, _):
                buf0[i, pl.ds(j * 16, 16)] = jnp.zeros((16,), jnp.float32)
                return 0
            lax.fori_loop(0, D // 16, zb2, 0)
            return 0
        lax.fori_loop(0, CC, zb, 0)
        rbase = s * RPT
        for q in range(RPT // CC):
            pltpu.sync_copy(buf0, acc_sh.at[pl.ds(rbase + q * CC, CC)])
        zrem = RPT - (RPT // CC) * CC
        if zrem:
            pltpu.sync_copy(buf0.at[pl.ds(0, zrem)],
                            acc_sh.at[pl.ds(rbase + RPT - zrem, zrem)])

        ebase = wid * ET
        pltpu.sync_copy(row_hbm.at[pl.ds(ebase, ET)], rowb)

        SS = CC // 16
        SREM = CC - SS * 16

        def gstart(k, b, bufr, sem):
            pltpu.async_copy(xw_hbm.at[rowb.at[pl.ds(k * CC, CC)]], bufr, sem)
            pltpu.async_copy(col_hbm.at[pl.ds(ebase + k * CC, CC)],
                             colcb.at[b], sem)
            pltpu.async_copy(norm_hbm.at[pl.ds(ebase + k * CC, CC)],
                             normcb.at[b], sem)

        def gwait(k, b, bufr, sem):
            pltpu.make_async_copy(xw_hbm.at[rowb.at[pl.ds(k * CC, CC)]],
                                  bufr, sem).wait()
            pltpu.make_async_copy(col_hbm.at[pl.ds(ebase + k * CC, CC)],
                                  colcb.at[b], sem).wait()
            pltpu.make_async_copy(norm_hbm.at[pl.ds(ebase + k * CC, CC)],
                                  normcb.at[b], sem).wait()

        def scale(b, bufr):
            def mulrow(i, sc):
                for j in range(D // 16):
                    sl = pl.ds(j * 16, 16)
                    bufr[i, sl] = bufr[i, sl] * sc

            def edge16(g, _):
                nv = normcb[b, pl.ds(g * 16, 16)]
                for e in range(16):
                    mulrow(g * 16 + e, nv[e])
                return 0
            lax.fori_loop(0, SS, edge16, 0)
            if SREM:
                nv = normcb[b, pl.ds(CC - 16, 16)]
                for e in range(16 - SREM, 16):
                    mulrow(CC - 16 + e, nv[e])

        def scatter(b, bufr):
            pltpu.sync_copy(bufr, acc_sh.at[colcb.at[b]], add=True)

        assert NCH % 3 == 0
        bufs = ((buf0, sem0), (buf1, sem1), (buf2, sem2))
        for u in range(3):
            gstart(u, u, bufs[u][0], bufs[u][1])
        plsc.subcore_barrier()

        def triple(t, _):
            k0 = 3 * t
            for u in range(3):
                bufr, sem = bufs[u]
                gwait(k0 + u, u, bufr, sem)
                scale(u, bufr)
                scatter(u, bufr)

                @pl.when(k0 + u + 3 < NCH)
                def _():
                    gstart(k0 + u + 3, u, bufr, sem)
            return 0
        lax.fori_loop(0, NCH // 3, triple, 0)

        plsc.subcore_barrier()
        pltpu.sync_copy(acc_sh.at[pl.ds(rbase, RPT)],
                        out_hbm.at[c, pl.ds(rbase, RPT)])

    return sc_edge


def _mm_body(x_ref, w_ref, o_ref):
    o_ref[...] = jnp.dot(x_ref[...], w_ref[...],
                         preferred_element_type=jnp.float32)


def _tc_matmul(xx, ww):
    n, d = xx.shape
    h = ww.shape[1]
    return pl.pallas_call(
        _mm_body,
        grid=(n // BLK,),
        in_specs=[pl.BlockSpec((BLK, d), lambda i: (i, 0)),
                  pl.BlockSpec((d, h), lambda i: (0, 0))],
        out_specs=pl.BlockSpec((BLK, h), lambda i: (i, 0)),
        out_shape=jax.ShapeDtypeStruct((n, h), jnp.float32),
    )(xx, ww)


def _tc2_body(acc_ref, b_ref, w_ref, o_ref):
    t = acc_ref[0] + acc_ref[1] + b_ref[...]
    h = jnp.where(t > 0, t, jnp.exp(jnp.minimum(t, 0.0)) - 1.0)
    o_ref[...] = jnp.dot(h, w_ref[...], preferred_element_type=jnp.float32)


def _tc2(acc, b, ww):
    return pl.pallas_call(
        _tc2_body,
        grid=(N // BLK,),
        in_specs=[pl.BlockSpec((NC, BLK, D), lambda i: (0, i, 0)),
                  pl.BlockSpec((1, D), lambda i: (0, 0)),
                  pl.BlockSpec((D, D), lambda i: (0, 0))],
        out_specs=pl.BlockSpec((BLK, D), lambda i: (i, 0)),
        out_shape=jax.ShapeDtypeStruct((N, D), jnp.float32),
    )(acc, b, ww)


def _tc3_body(acc_ref, b_ref, batch_ref, fcw_ref, fcb_ref, o_ref, psum, pcnt):
    i = pl.program_id(0)

    @pl.when(i == 0)
    def _():
        psum[...] = jnp.zeros_like(psum)
        pcnt[...] = jnp.zeros_like(pcnt)

    t = acc_ref[0] + acc_ref[1] + b_ref[...]
    gids = lax.broadcasted_iota(jnp.int32, (G, BLK), 0)
    onehot = (gids == batch_ref[0]).astype(jnp.float32)
    psum[...] += jnp.dot(onehot, t, preferred_element_type=jnp.float32)
    pcnt[...] += jnp.sum(onehot, axis=1, keepdims=True)

    @pl.when(i == pl.num_programs(0) - 1)
    def _():
        pooled = psum[...] / jnp.maximum(pcnt[...], 1.0)
        o_ref[...] = (jnp.dot(pooled, fcw_ref[...],
                              preferred_element_type=jnp.float32)
                      + fcb_ref[...])


def _tc3(acc, b, batch2, fcwp, fcbp):
    return pl.pallas_call(
        _tc3_body,
        grid=(N // BLK,),
        in_specs=[pl.BlockSpec((NC, BLK, D), lambda i: (0, i, 0)),
                  pl.BlockSpec((1, D), lambda i: (0, 0)),
                  pl.BlockSpec((1, 1, BLK), lambda i: (i, 0, 0)),
                  pl.BlockSpec((D, D), lambda i: (0, 0)),
                  pl.BlockSpec((1, D), lambda i: (0, 0))],
        out_specs=pl.BlockSpec((G, D), lambda i: (0, 0)),
        out_shape=jax.ShapeDtypeStruct((G, D), jnp.float32),
        scratch_shapes=[pltpu.VMEM((G, D), jnp.float32),
                        pltpu.VMEM((G, 1), jnp.float32)],
    )(acc, b, batch2, fcwp, fcbp)


def kernel(x, edge_index, edge_weight, batch, W1, b1, W2, b2, fc_W, fc_b):
    row = edge_index[0].astype(jnp.int32)
    col = edge_index[1].astype(jnp.int32)
    loop = jnp.arange(N, dtype=jnp.int32)
    padi = jnp.zeros((EP - EL,), jnp.int32)
    row_f = jnp.concatenate([row, loop, padi])
    col_f = jnp.concatenate([col, loop, padi])
    ew_f = jnp.concatenate([edge_weight.astype(jnp.float32),
                            jnp.ones((N,), jnp.float32),
                            jnp.zeros((EP - EL,), jnp.float32)])
    sc_prep = _build_sc_prep()
    sc_edge = _build_sc_edge()

    norm = sc_prep(row_f, col_f, ew_f)
    xw1 = _tc_matmul(x, W1)
    acc1 = sc_edge(xw1, row_f, col_f, norm)
    xw2 = _tc2(acc1, b1.reshape(1, D), W2)
    acc2 = sc_edge(xw2, row_f, col_f, norm)

    fcwp = jnp.pad(fc_W, ((0, 0), (0, D - 1)))
    fcbp = jnp.pad(fc_b.reshape(1, 1), ((0, 0), (0, D - 1)))
    out = _tc3(acc2, b2.reshape(1, D),
               batch.astype(jnp.int32).reshape(N // BLK, 1, BLK), fcwp, fcbp)
    return out[:, :1]
